# baseline (speedup 1.0000x reference)
import functools

import jax
import jax.numpy as jnp
from jax import lax
from jax.experimental import pallas as pl
from jax.experimental.pallas import tpu as pltpu
from jax.experimental.pallas import tpu_sc as plsc

N = 10000
E = 320000
F_IN = 128
DIM = 32
C = 10

NC = 2
NS = 16
NW = NC * NS
CH = 128
KMAX = 80
E_PAD = NW * KMAX * CH
NPAD = E_PAD - E
N_ACC = 10240
RPT = N_ACC // NS
NPK = 4
NROW = N_ACC // NPK
DEGW = 8
DGRP = 8

_mesh = plsc.VectorSubcoreMesh(core_axis_name="c", subcore_axis_name="s")
_sc_params = pltpu.CompilerParams(use_tc_tiling_on_sc=False,
                                  needs_layout_passes=False)


@functools.partial(
    pl.kernel,
    out_type=jax.ShapeDtypeStruct((NC, N_ACC, DEGW), jnp.float32),
    mesh=_mesh,
    scratch_types=[
        pltpu.VMEM_SHARED((N_ACC, DEGW), jnp.float32),
        pltpu.VMEM((CH, DEGW), jnp.float32),
        pltpu.VMEM((KMAX, CH), jnp.int32),
        pltpu.SemaphoreType.DMA,
    ],
    compiler_params=_sc_params,
)
def _deg_kernel(dst_hbm, zeros_hbm, ones_hbm, out_hbm, shared, ones_v, dstb, sem):
    cid = lax.axis_index("c")
    sid = lax.axis_index("s")
    wid = sid * NC + cid
    row0 = sid * RPT

    pltpu.sync_copy(ones_hbm, ones_v)
    pltpu.sync_copy(dst_hbm.at[pl.ds(wid * KMAX, KMAX)], dstb)
    pltpu.sync_copy(zeros_hbm, shared.at[pl.ds(row0, RPT)])
    plsc.subcore_barrier()

    def group(g, carry):
        for b in range(DGRP):
            pltpu.async_copy(ones_v, shared.at[dstb.at[g * DGRP + b]], sem,
                             add=True)
        for b in range(DGRP):
            pltpu.make_async_copy(ones_v, shared.at[dstb.at[0]], sem).wait()
        return carry

    lax.fori_loop(0, KMAX // DGRP, group, None)
    plsc.subcore_barrier()
    pltpu.sync_copy(
        shared.at[pl.ds(row0, RPT)],
        out_hbm.at[cid, pl.ds(row0, RPT)],
    )


@functools.partial(
    pl.kernel,
    out_type=jax.ShapeDtypeStruct((NC, N_ACC, DIM), jnp.float32),
    mesh=_mesh,
    scratch_types=[
        pltpu.VMEM_SHARED((N_ACC, DIM), jnp.float32),
        pltpu.VMEM_SHARED((N_ACC, DIM), jnp.float32),
        pltpu.VMEM((KMAX, CH), jnp.int32),
        pltpu.VMEM((KMAX, CH), jnp.int32),
        pltpu.VMEM((RPT, DIM), jnp.float32),
        pltpu.VMEM((RPT,), jnp.float32),
        pltpu.VMEM((RPT, DEGW), jnp.float32),
        pltpu.VMEM((RPT, DEGW), jnp.float32),
        [pltpu.VMEM((CH, DIM), jnp.float32)] * 4,
        [pltpu.SemaphoreType.DMA] * 4,
        [pltpu.SemaphoreType.DMA] * 4,
    ],
    compiler_params=_sc_params,
)
def _agg_kernel(z_hbm, degp_hbm, src_hbm, dst_hbm, out_hbm,
                shared, zs_sp, srcb, dstb, zbuf, dbuf, p0b, p1b,
                rows, gsem, ssem):
    cid = lax.axis_index("c")
    sid = lax.axis_index("s")
    wid = sid * NC + cid
    row0 = sid * RPT

    pltpu.sync_copy(src_hbm.at[pl.ds(wid * KMAX, KMAX)], srcb)
    pltpu.sync_copy(dst_hbm.at[pl.ds(wid * KMAX, KMAX)], dstb)
    pltpu.sync_copy(z_hbm.at[pl.ds(row0, RPT)], zbuf)
    pltpu.sync_copy(degp_hbm.at[0, pl.ds(row0, RPT)], p0b)
    pltpu.sync_copy(degp_hbm.at[1, pl.ds(row0, RPT)], p1b)

    def dcalc(i, carry):
        ridx = lax.iota(jnp.int32, 16) + i * 16
        cidx = jnp.zeros((16,), jnp.int32)
        g0 = plsc.load_gather(p0b, [ridx, cidx])
        g1 = plsc.load_gather(p1b, [ridx, cidx])
        node = lax.iota(jnp.int32, 16) + (row0 + i * 16)
        corr = jnp.where(node < NPAD, 1.0, 0.0).astype(jnp.float32)
        v = g0 + g1 + 2.0 - corr
        y = plsc.bitcast(
            jnp.int32(0x5F3759DF) - (plsc.bitcast(v, jnp.int32) >> 1),
            jnp.float32)
        for _ in range(3):
            y = y * (1.5 - 0.5 * v * y * y)
        dbuf[pl.ds(i * 16, 16)] = y
        return carry

    lax.fori_loop(0, RPT // 16, dcalc, None)

    def dscale(i, carry):
        ridx = lax.iota(jnp.int32, 16) + i * 16
        dvec = dbuf[pl.ds(i * 16, 16)]
        for c in range(DIM):
            cidx = jnp.full((16,), c, jnp.int32)
            v = plsc.load_gather(zbuf, [ridx, cidx]) * dvec
            plsc.store_scatter(zbuf, [ridx, cidx], v)
        return carry

    lax.fori_loop(0, RPT // 16, dscale, None)
    pltpu.sync_copy(zbuf, zs_sp.at[pl.ds(row0, RPT)])

    @pl.when(cid == 0)
    def _():
        def dbl(n, carry):
            zbuf[n, pl.ds(0, 16)] = zbuf[n, pl.ds(0, 16)] * 2.0
            zbuf[n, pl.ds(16, 16)] = zbuf[n, pl.ds(16, 16)] * 2.0
            return carry

        lax.fori_loop(0, RPT, dbl, None)

    @pl.when(cid != 0)
    def _():
        def zero(n, carry):
            zbuf[n, pl.ds(0, 16)] = jnp.zeros((16,), jnp.float32)
            zbuf[n, pl.ds(16, 16)] = jnp.zeros((16,), jnp.float32)
            return carry

        lax.fori_loop(0, RPT, zero, None)

    pltpu.sync_copy(zbuf, shared.at[pl.ds(row0, RPT)])
    plsc.subcore_barrier()

    def gwait(i):
        pltpu.make_async_copy(zs_sp.at[srcb.at[0]], rows[i], gsem[i]).wait()

    def swait(i):
        pltpu.make_async_copy(rows[i], shared.at[dstb.at[0]], ssem[i]).wait()

    pltpu.async_copy(zs_sp.at[srcb.at[0]], rows[0], gsem[0])
    pltpu.async_copy(zs_sp.at[srcb.at[1]], rows[1], gsem[1])

    def body(j, carry):
        k0 = 4 * j
        for i in range(4):
            k = k0 + i
            gwait(i)
            pltpu.async_copy(rows[i], shared.at[dstb.at[k]], ssem[i], add=True)
            i2 = (i + 2) % 4

            @pl.when(k + 2 < KMAX)
            def _():
                @pl.when(k >= 2)
                def _():
                    swait(i2)
                pltpu.async_copy(zs_sp.at[srcb.at[k + 2]], rows[i2], gsem[i2])

        return carry

    lax.fori_loop(0, KMAX // 4, body, None)
    for i in range(4):
        swait((KMAX - 4 + i) % 4)
    plsc.subcore_barrier()

    pltpu.sync_copy(shared.at[pl.ds(row0, RPT)], zbuf)

    def oscale(i, carry):
        ridx = lax.iota(jnp.int32, 16) + i * 16
        dvec = dbuf[pl.ds(i * 16, 16)]
        for c in range(DIM):
            cidx = jnp.full((16,), c, jnp.int32)
            v = plsc.load_gather(zbuf, [ridx, cidx]) * dvec
            plsc.store_scatter(zbuf, [ridx, cidx], v)
        return carry

    lax.fori_loop(0, RPT // 16, oscale, None)
    pltpu.sync_copy(zbuf, out_hbm.at[cid, pl.ds(row0, RPT)])


def _tc0_body(x4_ref, w_ref, zp_ref):
    zp_ref[...] = jnp.dot(x4_ref[...], w_ref[...],
                          preferred_element_type=jnp.float32)


def _tc_mid_body(aggp_ref, b_ref, w_ref, zp_ref):
    hp = jnp.maximum(aggp_ref[0] + aggp_ref[1] + b_ref[...], 0.0)
    zp = jnp.dot(hp, w_ref[...], preferred_element_type=jnp.float32)
    zp_ref[...] = zp
    zp_ref[N // NPK:NROW] = jnp.zeros((NROW - N // NPK, NPK * DIM), jnp.float32)


def _tc_out_body(aggp_ref, b_ref, wl_ref, bl_ref, g_ref, out_ref):
    hp = jnp.maximum(aggp_ref[0] + aggp_ref[1] + b_ref[...], 0.0)
    lo = jnp.dot(hp, wl_ref[...], preferred_element_type=jnp.float32) + bl_ref[...]
    e = jnp.exp(lo)
    s = jnp.dot(e, g_ref[...], preferred_element_type=jnp.float32)
    out_ref[...] = e / s


def kernel(x, edge_index, W1, b1, W2, b2, Wl, bl):
    src = edge_index[0].astype(jnp.int32)
    dst = edge_index[1].astype(jnp.int32)
    src_p = jnp.concatenate([src, jnp.full((NPAD,), N, jnp.int32)])
    dst_p = jnp.concatenate(
        [dst, jnp.arange(NPAD, dtype=jnp.int32) % N_ACC])
    src_p = src_p.reshape(NW * KMAX, CH)
    dst_p = dst_p.reshape(NW * KMAX, CH)

    zeros8 = jnp.zeros((RPT, DEGW), jnp.float32)
    ones8 = jnp.ones((CH, DEGW), jnp.float32)

    eye = jnp.eye(NPK, dtype=jnp.float32)
    W1bd = jnp.kron(eye, W1)
    W2bd = jnp.kron(eye, W2)
    Wlpad = jnp.zeros((DIM, DIM), jnp.float32).at[:, :C].set(Wl)
    Wlbd = jnp.kron(eye, Wlpad)
    blpad = jnp.full((DIM,), -1e30, jnp.float32).at[:C].set(bl)
    blp = jnp.tile(blpad, NPK).reshape(1, NPK * DIM)
    b1p = jnp.tile(b1, NPK).reshape(1, NPK * DIM)
    b2p = jnp.tile(b2, NPK).reshape(1, NPK * DIM)
    G = jnp.kron(eye, jnp.ones((DIM, DIM), jnp.float32))

    x4 = jnp.pad(x, ((0, N_ACC - N), (0, 0))).reshape(NROW, NPK * F_IN)

    degp = _deg_kernel(dst_p, zeros8, ones8)

    z1p = pl.pallas_call(
        _tc0_body,
        out_shape=jax.ShapeDtypeStruct((NROW, NPK * DIM), jnp.float32),
    )(x4, W1bd)

    agg1 = _agg_kernel(z1p.reshape(N_ACC, DIM), degp, src_p, dst_p)

    z2p = pl.pallas_call(
        _tc_mid_body,
        out_shape=jax.ShapeDtypeStruct((NROW, NPK * DIM), jnp.float32),
    )(agg1.reshape(NC, NROW, NPK * DIM), b1p, W2bd)

    agg2 = _agg_kernel(z2p.reshape(N_ACC, DIM), degp, src_p, dst_p)

    outp = pl.pallas_call(
        _tc_out_body,
        out_shape=jax.ShapeDtypeStruct((NROW, NPK * DIM), jnp.float32),
    )(agg2.reshape(NC, NROW, NPK * DIM), b2p, Wlbd, blp, G)

    return outp.reshape(N_ACC, DIM)[:N, :C]


# speedup vs baseline: 1.6052x; 1.6052x over previous
import functools

import jax
import jax.numpy as jnp
from jax import lax
from jax.experimental import pallas as pl
from jax.experimental.pallas import tpu as pltpu
from jax.experimental.pallas import tpu_sc as plsc

N = 10000
E = 320000
F_IN = 128
DIM = 32
C = 10

NC = 2
NS = 16
NW = NC * NS
CH = 128
KMAX = 80
E_PAD = NW * KMAX * CH
NPAD = E_PAD - E
N_ACC = 10240
RPT = N_ACC // NS
NPK = 4
NROW = N_ACC // NPK
DEGW = 8
DGRP = 8

_mesh = plsc.VectorSubcoreMesh(core_axis_name="c", subcore_axis_name="s")
_sc_params = pltpu.CompilerParams(use_tc_tiling_on_sc=False,
                                  needs_layout_passes=False)


@functools.partial(
    pl.kernel,
    out_type=jax.ShapeDtypeStruct((NC, N_ACC, DEGW), jnp.float32),
    mesh=_mesh,
    scratch_types=[
        pltpu.VMEM_SHARED((N_ACC, DEGW), jnp.float32),
        pltpu.VMEM((CH, DEGW), jnp.float32),
        pltpu.VMEM((KMAX, CH), jnp.int32),
        pltpu.SemaphoreType.DMA,
    ],
    compiler_params=_sc_params,
)
def _deg_kernel(dst_hbm, zeros_hbm, ones_hbm, out_hbm, shared, ones_v, dstb, sem):
    cid = lax.axis_index("c")
    sid = lax.axis_index("s")
    wid = sid * NC + cid
    row0 = sid * RPT

    pltpu.sync_copy(ones_hbm, ones_v)
    pltpu.sync_copy(dst_hbm.at[pl.ds(wid * KMAX, KMAX)], dstb)
    pltpu.sync_copy(zeros_hbm, shared.at[pl.ds(row0, RPT)])
    plsc.subcore_barrier()

    def group(g, carry):
        for b in range(DGRP):
            pltpu.async_copy(ones_v, shared.at[dstb.at[g * DGRP + b]], sem,
                             add=True)
        for b in range(DGRP):
            pltpu.make_async_copy(ones_v, shared.at[dstb.at[0]], sem).wait()
        return carry

    lax.fori_loop(0, KMAX // DGRP, group, None)
    plsc.subcore_barrier()
    pltpu.sync_copy(
        shared.at[pl.ds(row0, RPT)],
        out_hbm.at[cid, pl.ds(row0, RPT)],
    )


@functools.partial(
    pl.kernel,
    out_type=jax.ShapeDtypeStruct((NC, N_ACC, DIM), jnp.float32),
    mesh=_mesh,
    scratch_types=[
        pltpu.VMEM_SHARED((N_ACC, DIM), jnp.float32),
        pltpu.VMEM_SHARED((N_ACC, DIM), jnp.float32),
        pltpu.VMEM((KMAX, CH), jnp.int32),
        pltpu.VMEM((KMAX, CH), jnp.int32),
        pltpu.VMEM((RPT, DIM), jnp.float32),
        pltpu.VMEM((RPT,), jnp.float32),
        pltpu.VMEM((RPT, DEGW), jnp.float32),
        pltpu.VMEM((RPT, DEGW), jnp.float32),
        [pltpu.VMEM((CH, DIM), jnp.float32)] * 4,
        [pltpu.SemaphoreType.DMA] * 4,
        [pltpu.SemaphoreType.DMA] * 4,
    ],
    compiler_params=_sc_params,
)
def _agg_kernel(z_hbm, degp_hbm, src_hbm, dst_hbm, out_hbm,
                shared, zs_sp, srcb, dstb, zbuf, dbuf, p0b, p1b,
                rows, gsem, ssem):
    cid = lax.axis_index("c")
    sid = lax.axis_index("s")
    wid = sid * NC + cid
    row0 = sid * RPT

    pltpu.sync_copy(src_hbm.at[pl.ds(wid * KMAX, KMAX)], srcb)
    pltpu.sync_copy(dst_hbm.at[pl.ds(wid * KMAX, KMAX)], dstb)
    pltpu.sync_copy(z_hbm.at[pl.ds(row0, RPT)], zbuf)
    pltpu.sync_copy(degp_hbm.at[0, pl.ds(row0, RPT)], p0b)
    pltpu.sync_copy(degp_hbm.at[1, pl.ds(row0, RPT)], p1b)

    def dcalc(i, carry):
        ridx = lax.iota(jnp.int32, 16) + i * 16
        cidx = jnp.zeros((16,), jnp.int32)
        g0 = plsc.load_gather(p0b, [ridx, cidx])
        g1 = plsc.load_gather(p1b, [ridx, cidx])
        node = lax.iota(jnp.int32, 16) + (row0 + i * 16)
        corr = jnp.where(node < NPAD, 1.0, 0.0).astype(jnp.float32)
        v = g0 + g1 + 2.0 - corr
        y = plsc.bitcast(
            jnp.int32(0x5F3759DF) - (plsc.bitcast(v, jnp.int32) >> 1),
            jnp.float32)
        for _ in range(3):
            y = y * (1.5 - 0.5 * v * y * y)
        dbuf[pl.ds(i * 16, 16)] = y
        return carry

    lax.fori_loop(0, RPT // 16, dcalc, None)

    def dscale(i, carry):
        base = i * 16
        dvec = dbuf[pl.ds(base, 16)]
        for r in range(16):
            dn = dvec[r]
            zbuf[base + r, pl.ds(0, 16)] = zbuf[base + r, pl.ds(0, 16)] * dn
            zbuf[base + r, pl.ds(16, 16)] = zbuf[base + r, pl.ds(16, 16)] * dn
        return carry

    lax.fori_loop(0, RPT // 16, dscale, None)
    pltpu.sync_copy(zbuf, zs_sp.at[pl.ds(row0, RPT)])

    @pl.when(cid == 0)
    def _():
        def dbl(n, carry):
            zbuf[n, pl.ds(0, 16)] = zbuf[n, pl.ds(0, 16)] * 2.0
            zbuf[n, pl.ds(16, 16)] = zbuf[n, pl.ds(16, 16)] * 2.0
            return carry

        lax.fori_loop(0, RPT, dbl, None)

    @pl.when(cid != 0)
    def _():
        def zero(n, carry):
            zbuf[n, pl.ds(0, 16)] = jnp.zeros((16,), jnp.float32)
            zbuf[n, pl.ds(16, 16)] = jnp.zeros((16,), jnp.float32)
            return carry

        lax.fori_loop(0, RPT, zero, None)

    pltpu.sync_copy(zbuf, shared.at[pl.ds(row0, RPT)])
    plsc.subcore_barrier()

    def gwait(i):
        pltpu.make_async_copy(zs_sp.at[srcb.at[0]], rows[i], gsem[i]).wait()

    def swait(i):
        pltpu.make_async_copy(rows[i], shared.at[dstb.at[0]], ssem[i]).wait()

    pltpu.async_copy(zs_sp.at[srcb.at[0]], rows[0], gsem[0])
    pltpu.async_copy(zs_sp.at[srcb.at[1]], rows[1], gsem[1])

    def body(j, carry):
        k0 = 4 * j
        for i in range(4):
            k = k0 + i
            gwait(i)
            pltpu.async_copy(rows[i], shared.at[dstb.at[k]], ssem[i], add=True)
            i2 = (i + 2) % 4

            @pl.when(k + 2 < KMAX)
            def _():
                @pl.when(k >= 2)
                def _():
                    swait(i2)
                pltpu.async_copy(zs_sp.at[srcb.at[k + 2]], rows[i2], gsem[i2])

        return carry

    lax.fori_loop(0, KMAX // 4, body, None)
    for i in range(4):
        swait((KMAX - 4 + i) % 4)
    plsc.subcore_barrier()

    pltpu.sync_copy(shared.at[pl.ds(row0, RPT)], zbuf)

    def oscale(i, carry):
        base = i * 16
        dvec = dbuf[pl.ds(base, 16)]
        for r in range(16):
            dn = dvec[r]
            zbuf[base + r, pl.ds(0, 16)] = zbuf[base + r, pl.ds(0, 16)] * dn
            zbuf[base + r, pl.ds(16, 16)] = zbuf[base + r, pl.ds(16, 16)] * dn
        return carry

    lax.fori_loop(0, RPT // 16, oscale, None)
    pltpu.sync_copy(zbuf, out_hbm.at[cid, pl.ds(row0, RPT)])


def _tc0_body(x4_ref, w_ref, zp_ref):
    zp_ref[...] = jnp.dot(x4_ref[...], w_ref[...],
                          preferred_element_type=jnp.float32)


def _tc_mid_body(aggp_ref, b_ref, w_ref, zp_ref):
    hp = jnp.maximum(aggp_ref[0] + aggp_ref[1] + b_ref[...], 0.0)
    zp = jnp.dot(hp, w_ref[...], preferred_element_type=jnp.float32)
    zp_ref[...] = zp
    zp_ref[N // NPK:NROW] = jnp.zeros((NROW - N // NPK, NPK * DIM), jnp.float32)


def _tc_out_body(aggp_ref, b_ref, wl_ref, bl_ref, g_ref, out_ref):
    hp = jnp.maximum(aggp_ref[0] + aggp_ref[1] + b_ref[...], 0.0)
    lo = jnp.dot(hp, wl_ref[...], preferred_element_type=jnp.float32) + bl_ref[...]
    e = jnp.exp(lo)
    s = jnp.dot(e, g_ref[...], preferred_element_type=jnp.float32)
    out_ref[...] = e / s


def kernel(x, edge_index, W1, b1, W2, b2, Wl, bl):
    src = edge_index[0].astype(jnp.int32)
    dst = edge_index[1].astype(jnp.int32)
    src_p = jnp.concatenate([src, jnp.full((NPAD,), N, jnp.int32)])
    dst_p = jnp.concatenate(
        [dst, jnp.arange(NPAD, dtype=jnp.int32) % N_ACC])
    src_p = src_p.reshape(NW * KMAX, CH)
    dst_p = dst_p.reshape(NW * KMAX, CH)

    zeros8 = jnp.zeros((RPT, DEGW), jnp.float32)
    ones8 = jnp.ones((CH, DEGW), jnp.float32)

    eye = jnp.eye(NPK, dtype=jnp.float32)
    W1bd = jnp.kron(eye, W1)
    W2bd = jnp.kron(eye, W2)
    Wlpad = jnp.zeros((DIM, DIM), jnp.float32).at[:, :C].set(Wl)
    Wlbd = jnp.kron(eye, Wlpad)
    blpad = jnp.full((DIM,), -1e30, jnp.float32).at[:C].set(bl)
    blp = jnp.tile(blpad, NPK).reshape(1, NPK * DIM)
    b1p = jnp.tile(b1, NPK).reshape(1, NPK * DIM)
    b2p = jnp.tile(b2, NPK).reshape(1, NPK * DIM)
    G = jnp.kron(eye, jnp.ones((DIM, DIM), jnp.float32))

    x4 = jnp.pad(x, ((0, N_ACC - N), (0, 0))).reshape(NROW, NPK * F_IN)

    degp = _deg_kernel(dst_p, zeros8, ones8)

    z1p = pl.pallas_call(
        _tc0_body,
        out_shape=jax.ShapeDtypeStruct((NROW, NPK * DIM), jnp.float32),
    )(x4, W1bd)

    agg1 = _agg_kernel(z1p.reshape(N_ACC, DIM), degp, src_p, dst_p)

    z2p = pl.pallas_call(
        _tc_mid_body,
        out_shape=jax.ShapeDtypeStruct((NROW, NPK * DIM), jnp.float32),
    )(agg1.reshape(NC, NROW, NPK * DIM), b1p, W2bd)

    agg2 = _agg_kernel(z2p.reshape(N_ACC, DIM), degp, src_p, dst_p)

    outp = pl.pallas_call(
        _tc_out_body,
        out_shape=jax.ShapeDtypeStruct((NROW, NPK * DIM), jnp.float32),
    )(agg2.reshape(NC, NROW, NPK * DIM), b2p, Wlbd, blp, G)

    return outp.reshape(N_ACC, DIM)[:N, :C]
